# baseline (speedup 1.0000x reference)
import functools

import jax
import jax.numpy as jnp
from jax import lax
from jax.experimental import pallas as pl
from jax.experimental.pallas import tpu as pltpu
from jax.experimental.pallas import tpu_sc as plsc

EMBED_D = 128
N_LAYERS = 2
N_TYPES = 3
NODE_COUNT = 100000
MIN_SIZE = 10
BATCH = 1024
N_TRIPLE = 3
N_CHAIN = N_LAYERS * N_TYPES
N_PAIR = N_CHAIN // 2
D2 = 2 * EMBED_D

NEIGH_ROWS = N_TRIPLE * N_CHAIN * BATCH * MIN_SIZE
CENT_ROWS = N_TRIPLE * BATCH

NC = 2
NS = 16
NW = NC * NS
CHUNK = 128
N_PER_W = NEIGH_ROWS // NW
NCH = N_PER_W // CHUNK
C_PER_W = CENT_ROWS // NW


def _sc_gather(tab_flat, idx_c, idx_n):
    mesh = plsc.VectorSubcoreMesh(core_axis_name="c", subcore_axis_name="s")

    @functools.partial(
        pl.kernel,
        mesh=mesh,
        out_type=(
            jax.ShapeDtypeStruct((CENT_ROWS, EMBED_D), jnp.float32),
            jax.ShapeDtypeStruct((NEIGH_ROWS, EMBED_D), jnp.float32),
        ),
        scratch_types=[
            pltpu.VMEM((C_PER_W,), jnp.int32),
            pltpu.VMEM((C_PER_W, EMBED_D), jnp.float32),
            pltpu.VMEM((CHUNK,), jnp.int32),
            pltpu.VMEM((CHUNK,), jnp.int32),
            pltpu.VMEM((CHUNK, EMBED_D), jnp.float32),
            pltpu.VMEM((CHUNK, EMBED_D), jnp.float32),
            pltpu.SemaphoreType.DMA,
            pltpu.SemaphoreType.DMA,
            pltpu.SemaphoreType.DMA,
        ],
    )
    def k(tab, ic, inn, oc, on, civ, crv, iv0, iv1, rv0, rv1, sem_c, sem0, sem1):
        wid = lax.axis_index("s") * NC + lax.axis_index("c")
        cbase = wid * C_PER_W
        pltpu.sync_copy(ic.at[pl.ds(cbase, C_PER_W)], civ)
        ccp = pltpu.async_copy(tab.at[civ], crv, sem_c)

        nbase = wid * N_PER_W
        ivs = (iv0, iv1)
        rvs = (rv0, rv1)
        sems = (sem0, sem1)
        pltpu.sync_copy(inn.at[pl.ds(nbase, CHUNK)], ivs[0])
        cps = [pltpu.async_copy(tab.at[ivs[0]], rvs[0], sems[0]), None]
        for i in range(NCH):
            b = i % 2
            nb = (i + 1) % 2
            if i + 1 < NCH:
                pltpu.sync_copy(inn.at[pl.ds(nbase + (i + 1) * CHUNK, CHUNK)], ivs[nb])
                cps[nb] = pltpu.async_copy(tab.at[ivs[nb]], rvs[nb], sems[nb])
            cps[b].wait()
            pltpu.sync_copy(rvs[b], on.at[pl.ds(nbase + i * CHUNK, CHUNK)])
        ccp.wait()
        pltpu.sync_copy(crv, oc.at[pl.ds(cbase, C_PER_W)])

    return k(tab_flat, idx_c, idx_n)


_DN = (((1,), (1,)), ((), ()))
_PREC = lax.Precision.DEFAULT


def _tc_body(nref, cref, wibd, whbd, bbd, sw, oref):
    bb = cref.shape[1]
    cur = cref[0]

    pres = []
    for p in range(N_PAIR):
        x2 = nref[0, p].reshape(MIN_SIZE * bb, D2)
        pre = lax.dot_general(x2, wibd[p], _DN,
                              preferred_element_type=jnp.float32,
                              precision=_PREC)
        pre = pre + bbd[p][None, :]
        pres.append(pre.reshape(MIN_SIZE, bb, D2))

    hs = [jnp.zeros((bb, D2), jnp.float32) for _ in range(N_PAIR)]
    accs = [jnp.zeros((bb, D2), jnp.float32) for _ in range(N_PAIR)]
    for s in range(MIN_SIZE):
        for p in range(N_PAIR):
            hp = jnp.tanh(pres[p][s]
                          + lax.dot_general(hs[p], whbd[p], _DN,
                                            preferred_element_type=jnp.float32,
                                            precision=_PREC))
            hs[p] = hp
            accs[p] = accs[p] + hp

    inv = jnp.float32(1.0 / MIN_SIZE)
    for l in range(N_LAYERS):
        w1 = sw[0, l:l + 1, 0:EMBED_D]
        w2 = sw[0, l:l + 1, EMBED_D:2 * EMBED_D]
        base = jnp.sum(cur * w1, axis=1, keepdims=True)
        aggs = []
        for t in range(N_TYPES):
            k = l * N_TYPES + t
            ph, hh = divmod(k, 2)
            aggs.append(accs[ph][:, hh * EMBED_D:(hh + 1) * EMBED_D] * inv)
        logits = [base + jnp.sum(cur * w2, axis=1, keepdims=True)]
        logits += [base + jnp.sum(a * w2, axis=1, keepdims=True) for a in aggs]
        m = jnp.maximum(jnp.maximum(logits[0], logits[1]),
                        jnp.maximum(logits[2], logits[3]))
        es = [jnp.exp(x - m) for x in logits]
        den = es[0] + es[1] + es[2] + es[3]
        new = es[0] * cur
        for t in range(N_TYPES):
            new = new + es[1 + t] * aggs[t]
        new = new / den
        cur = jnp.where(new >= 0, new, jnp.float32(0.01) * new)

    oref[0] = cur


def _tc_compute(neigh, cent, W_ibd, W_hbd, b_bd, sem_sel, bb=512):
    nb = BATCH // bb
    grid = (N_TRIPLE, nb)
    return pl.pallas_call(
        _tc_body,
        grid=grid,
        in_specs=[
            pl.BlockSpec((1, N_PAIR, MIN_SIZE, bb, D2),
                         lambda i, j: (i, 0, 0, j, 0)),
            pl.BlockSpec((1, bb, EMBED_D), lambda i, j: (i, j, 0)),
            pl.BlockSpec(W_ibd.shape, lambda i, j: (0, 0, 0)),
            pl.BlockSpec(W_hbd.shape, lambda i, j: (0, 0, 0)),
            pl.BlockSpec(b_bd.shape, lambda i, j: (0, 0)),
            pl.BlockSpec((1, N_LAYERS, 2 * EMBED_D), lambda i, j: (i, 0, 0)),
        ],
        out_specs=pl.BlockSpec((1, bb, EMBED_D), lambda i, j: (i, j, 0)),
        out_shape=jax.ShapeDtypeStruct((N_TRIPLE, BATCH, EMBED_D), jnp.float32),
        compiler_params=pltpu.CompilerParams(
            dimension_semantics=("parallel", "parallel")),
    )(neigh, cent, W_ibd, W_hbd, b_bd, sem_sel)


def _block_diag_pairs(W):
    out = jnp.zeros((N_PAIR, D2, D2), jnp.float32)
    out = out.at[:, :EMBED_D, :EMBED_D].set(W[0::2])
    out = out.at[:, EMBED_D:, EMBED_D:].set(W[1::2])
    return out


def kernel(c_idx, pos_idx, neg_idx, neigh_c, neigh_pos, neigh_neg,
           tables, W_ih, W_hh, b_ih, b_hh, sem_w):
    tab_flat = tables.reshape(N_TYPES * NODE_COUNT, EMBED_D)
    toff = (jnp.arange(N_TYPES, dtype=jnp.int32) * NODE_COUNT)[None, :, None, None]

    def flat_n(n):
        x = (n.astype(jnp.int32) + toff).reshape(N_PAIR, 2, BATCH, MIN_SIZE)
        return x.transpose(0, 3, 2, 1).reshape(-1)

    idx_c = jnp.concatenate([
        c_idx.astype(jnp.int32),
        pos_idx.astype(jnp.int32) + NODE_COUNT,
        neg_idx.astype(jnp.int32) + NODE_COUNT,
    ])
    idx_n = jnp.concatenate([flat_n(neigh_c), flat_n(neigh_pos), flat_n(neigh_neg)])

    cent, neigh = _sc_gather(tab_flat, idx_c, idx_n)
    cent = cent.reshape(N_TRIPLE, BATCH, EMBED_D)
    neigh = neigh.reshape(N_TRIPLE, N_PAIR, MIN_SIZE, BATCH, D2)

    W_ibd = _block_diag_pairs(W_ih.reshape(N_CHAIN, EMBED_D, EMBED_D))
    W_hbd = _block_diag_pairs(W_hh.reshape(N_CHAIN, EMBED_D, EMBED_D))
    b_bd = (b_ih + b_hh).reshape(N_PAIR, D2)

    sem_sel = jnp.stack([sem_w[:, 0], sem_w[:, 1], sem_w[:, 1]], axis=0)

    out = _tc_compute(neigh, cent, W_ibd, W_hbd, b_bd, sem_sel)
    return (out[0], out[1], out[2])


# speedup vs baseline: 1.0992x; 1.0992x over previous
import functools

import jax
import jax.numpy as jnp
from jax import lax
from jax.experimental import pallas as pl
from jax.experimental.pallas import tpu as pltpu
from jax.experimental.pallas import tpu_sc as plsc

EMBED_D = 128
N_LAYERS = 2
N_TYPES = 3
NODE_COUNT = 100000
MIN_SIZE = 10
BATCH = 1024
N_TRIPLE = 3
N_CHAIN = N_LAYERS * N_TYPES
N_PAIR = N_CHAIN // 2
D2 = 2 * EMBED_D

NEIGH_ROWS = N_CHAIN * BATCH * MIN_SIZE
CENT_ROWS = BATCH

NC = 2
NS = 16
NW = NC * NS
CHUNK = 128
N_PER_W = NEIGH_ROWS // NW
NCH = N_PER_W // CHUNK
C_PER_W = CENT_ROWS // NW


def _sc_gather(tab_flat, idx_c, idx_n):
    mesh = plsc.VectorSubcoreMesh(core_axis_name="c", subcore_axis_name="s")

    @functools.partial(
        pl.kernel,
        mesh=mesh,
        out_type=(
            jax.ShapeDtypeStruct((CENT_ROWS, EMBED_D), jnp.float32),
            jax.ShapeDtypeStruct((NEIGH_ROWS, EMBED_D), jnp.float32),
        ),
        scratch_types=[
            pltpu.VMEM((C_PER_W,), jnp.int32),
            pltpu.VMEM((C_PER_W, EMBED_D), jnp.float32),
            pltpu.VMEM((CHUNK,), jnp.int32),
            pltpu.VMEM((CHUNK,), jnp.int32),
            pltpu.VMEM((CHUNK, EMBED_D), jnp.float32),
            pltpu.VMEM((CHUNK, EMBED_D), jnp.float32),
            pltpu.SemaphoreType.DMA,
            pltpu.SemaphoreType.DMA,
            pltpu.SemaphoreType.DMA,
        ],
    )
    def k(tab, ic, inn, oc, on, civ, crv, iv0, iv1, rv0, rv1, sem_c, sem0, sem1):
        wid = lax.axis_index("s") * NC + lax.axis_index("c")
        cbase = wid * C_PER_W
        pltpu.sync_copy(ic.at[pl.ds(cbase, C_PER_W)], civ)
        ccp = pltpu.async_copy(tab.at[civ], crv, sem_c)

        nbase = wid * N_PER_W
        ivs = (iv0, iv1)
        rvs = (rv0, rv1)
        sems = (sem0, sem1)
        pltpu.sync_copy(inn.at[pl.ds(nbase, CHUNK)], ivs[0])
        cps = [pltpu.async_copy(tab.at[ivs[0]], rvs[0], sems[0]), None]
        for i in range(NCH):
            b = i % 2
            nb = (i + 1) % 2
            if i + 1 < NCH:
                pltpu.sync_copy(inn.at[pl.ds(nbase + (i + 1) * CHUNK, CHUNK)], ivs[nb])
                cps[nb] = pltpu.async_copy(tab.at[ivs[nb]], rvs[nb], sems[nb])
            cps[b].wait()
            pltpu.sync_copy(rvs[b], on.at[pl.ds(nbase + i * CHUNK, CHUNK)])
        ccp.wait()
        pltpu.sync_copy(crv, oc.at[pl.ds(cbase, C_PER_W)])

    return k(tab_flat, idx_c, idx_n)


_DN = (((1,), (1,)), ((), ()))
_PREC = lax.Precision.DEFAULT


def _tc_body(nref, cref, wibd, whbd, bbd, sw, oref):
    bb = cref.shape[0]
    cur = cref[...]

    pres = []
    for p in range(N_PAIR):
        x2 = nref[p].reshape(MIN_SIZE * bb, D2)
        pre = lax.dot_general(x2, wibd[p], _DN,
                              preferred_element_type=jnp.float32,
                              precision=_PREC)
        pre = pre + bbd[p][None, :]
        pres.append(pre.reshape(MIN_SIZE, bb, D2))

    hs = [jnp.zeros((bb, D2), jnp.float32) for _ in range(N_PAIR)]
    accs = [jnp.zeros((bb, D2), jnp.float32) for _ in range(N_PAIR)]
    for s in range(MIN_SIZE):
        for p in range(N_PAIR):
            hp = jnp.tanh(pres[p][s]
                          + lax.dot_general(hs[p], whbd[p], _DN,
                                            preferred_element_type=jnp.float32,
                                            precision=_PREC))
            hs[p] = hp
            accs[p] = accs[p] + hp

    inv = jnp.float32(1.0 / MIN_SIZE)
    for l in range(N_LAYERS):
        w1 = sw[0, l:l + 1, 0:EMBED_D]
        w2 = sw[0, l:l + 1, EMBED_D:2 * EMBED_D]
        base = jnp.sum(cur * w1, axis=1, keepdims=True)
        aggs = []
        for t in range(N_TYPES):
            k = l * N_TYPES + t
            ph, hh = divmod(k, 2)
            aggs.append(accs[ph][:, hh * EMBED_D:(hh + 1) * EMBED_D] * inv)
        logits = [base + jnp.sum(cur * w2, axis=1, keepdims=True)]
        logits += [base + jnp.sum(a * w2, axis=1, keepdims=True) for a in aggs]
        m = jnp.maximum(jnp.maximum(logits[0], logits[1]),
                        jnp.maximum(logits[2], logits[3]))
        es = [jnp.exp(x - m) for x in logits]
        den = es[0] + es[1] + es[2] + es[3]
        new = es[0] * cur
        for t in range(N_TYPES):
            new = new + es[1 + t] * aggs[t]
        new = new / den
        cur = jnp.where(new >= 0, new, jnp.float32(0.01) * new)

    oref[...] = cur


def _tc_compute(neigh, cent, W_ibd, W_hbd, b_bd, sem_t, bb=256):
    nb = BATCH // bb
    return pl.pallas_call(
        _tc_body,
        grid=(nb,),
        in_specs=[
            pl.BlockSpec((N_PAIR, MIN_SIZE, bb, D2), lambda j: (0, 0, j, 0)),
            pl.BlockSpec((bb, EMBED_D), lambda j: (j, 0)),
            pl.BlockSpec(W_ibd.shape, lambda j: (0, 0, 0)),
            pl.BlockSpec(W_hbd.shape, lambda j: (0, 0, 0)),
            pl.BlockSpec(b_bd.shape, lambda j: (0, 0)),
            pl.BlockSpec(sem_t.shape, lambda j: (0, 0, 0)),
        ],
        out_specs=pl.BlockSpec((bb, EMBED_D), lambda j: (j, 0)),
        out_shape=jax.ShapeDtypeStruct((BATCH, EMBED_D), jnp.float32),
        compiler_params=pltpu.CompilerParams(
            dimension_semantics=("parallel",)),
    )(neigh, cent, W_ibd, W_hbd, b_bd, sem_t)


def _block_diag_pairs(W):
    out = jnp.zeros((N_PAIR, D2, D2), jnp.float32)
    out = out.at[:, :EMBED_D, :EMBED_D].set(W[0::2])
    out = out.at[:, EMBED_D:, EMBED_D:].set(W[1::2])
    return out


def kernel(c_idx, pos_idx, neg_idx, neigh_c, neigh_pos, neigh_neg,
           tables, W_ih, W_hh, b_ih, b_hh, sem_w):
    tab_flat = tables.reshape(N_TYPES * NODE_COUNT, EMBED_D)
    toff = (jnp.arange(N_TYPES, dtype=jnp.int32) * NODE_COUNT)[None, :, None, None]

    def flat_n(n):
        x = (n.astype(jnp.int32) + toff).reshape(N_PAIR, 2, BATCH, MIN_SIZE)
        return x.transpose(0, 3, 2, 1).reshape(-1)

    W_ibd = _block_diag_pairs(W_ih.reshape(N_CHAIN, EMBED_D, EMBED_D))
    W_hbd = _block_diag_pairs(W_hh.reshape(N_CHAIN, EMBED_D, EMBED_D))
    b_bd = (b_ih + b_hh).reshape(N_PAIR, D2)

    triples = (
        (c_idx.astype(jnp.int32), flat_n(neigh_c), 0),
        (pos_idx.astype(jnp.int32) + NODE_COUNT, flat_n(neigh_pos), 1),
        (neg_idx.astype(jnp.int32) + NODE_COUNT, flat_n(neigh_neg), 1),
    )

    outs = []
    for idx_c, idx_n, ntype in triples:
        cent, neigh = _sc_gather(tab_flat, idx_c, idx_n)
        neigh = neigh.reshape(N_PAIR, MIN_SIZE, BATCH, D2)
        sem_t = sem_w[:, ntype][None]
        outs.append(_tc_compute(neigh, cent, W_ibd, W_hbd, b_bd, sem_t))
    return tuple(outs)
